# Initial kernel scaffold; baseline (speedup 1.0000x reference)
import jax
import jax.numpy as jnp
from jax.experimental import pallas as pl


def kernel(x, W_sy, W_tone, W_syllable_flag, W_ws):
    raise NotImplementedError("write your pallas kernel here")


# speedup vs baseline: 6.2359x; 6.2359x over previous
import functools

import jax
import jax.numpy as jnp
from jax import lax
from jax.experimental import pallas as pl
from jax.experimental.pallas import tpu as pltpu
from jax.experimental.pallas import tpu_sc as plsc

B, T = 1024, 200
VOCAB = 1024
D = 128
N = B * T
NC, NS = 2, 16
NW = NC * NS
TOK_PER_W = N // NW
C = 128
NCHUNK = TOK_PER_W // C

_mesh = plsc.VectorSubcoreMesh(core_axis_name="c", subcore_axis_name="s")


@functools.partial(
    pl.kernel,
    mesh=_mesh,
    out_type=jax.ShapeDtypeStruct((N, D), jnp.float32),
    scratch_types=[
        pltpu.VMEM((4, C), jnp.int32),
        pltpu.VMEM((C, D), jnp.float32),
        pltpu.VMEM((C, D), jnp.float32),
        pltpu.VMEM((C, D), jnp.float32),
        pltpu.VMEM((C, D), jnp.float32),
        pltpu.VMEM((C, D), jnp.float32),
        pltpu.SemaphoreType.DMA,
    ],
)
def _emb_sum_kernel(w_hbm, idx_hbm, out_hbm, idx_v, b0, b1, b2, b3, ov, sem):
    wid = lax.axis_index("s") * NC + lax.axis_index("c")
    base0 = wid * TOK_PER_W
    bufs = (b0, b1, b2, b3)

    def chunk_body(ci, carry):
        base = base0 + ci * C
        for k in range(4):
            pltpu.sync_copy(idx_hbm.at[k, pl.ds(base, C)], idx_v.at[k])
        cps = [pltpu.async_copy(w_hbm.at[idx_v.at[k]], bufs[k], sem)
               for k in range(4)]
        for cp in cps:
            cp.wait()

        def row_body(r, c2):
            for j in range(8):
                off = j * 16
                s0 = b0[r, pl.ds(off, 16)] + b1[r, pl.ds(off, 16)]
                s1 = b2[r, pl.ds(off, 16)] + b3[r, pl.ds(off, 16)]
                ov[r, pl.ds(off, 16)] = s0 + s1
            return c2

        lax.fori_loop(0, C, row_body, 0)
        pltpu.sync_copy(ov, out_hbm.at[pl.ds(base, C)])
        return carry

    lax.fori_loop(0, NCHUNK, chunk_body, 0)


def kernel(x, W_sy, W_tone, W_syllable_flag, W_ws):
    w_cat = jnp.concatenate([W_sy, W_tone, W_syllable_flag, W_ws], axis=0)
    idx = x.reshape(N, 4) + jnp.arange(4, dtype=jnp.int32) * VOCAB
    idx_all = idx.T
    out = _emb_sum_kernel(w_cat, idx_all)
    return out.reshape(B, T, D)


# speedup vs baseline: 10.9514x; 1.7562x over previous
import functools

import jax
import jax.numpy as jnp
from jax import lax
from jax.experimental import pallas as pl
from jax.experimental.pallas import tpu as pltpu
from jax.experimental.pallas import tpu_sc as plsc

B, T = 1024, 200
VOCAB = 1024
D = 128
N = B * T
NC, NS = 2, 16
NW = NC * NS
TOK_PER_W = N // NW
C = 64
NCHUNK = TOK_PER_W // C

_mesh = plsc.VectorSubcoreMesh(core_axis_name="c", subcore_axis_name="s")


@functools.partial(
    pl.kernel,
    mesh=_mesh,
    out_type=jax.ShapeDtypeStruct((N, D), jnp.float32),
    scratch_types=[
        pltpu.VMEM((4, TOK_PER_W), jnp.int32),
        pltpu.VMEM((C, D), jnp.float32),
        pltpu.VMEM((C, D), jnp.float32),
        pltpu.VMEM((C, D), jnp.float32),
        pltpu.VMEM((C, D), jnp.float32),
        pltpu.VMEM((C, D), jnp.float32),
        pltpu.VMEM((C, D), jnp.float32),
        pltpu.VMEM((C, D), jnp.float32),
        pltpu.VMEM((C, D), jnp.float32),
        pltpu.VMEM((C, D), jnp.float32),
        pltpu.VMEM((C, D), jnp.float32),
        pltpu.SemaphoreType.DMA,
        pltpu.SemaphoreType.DMA,
        pltpu.SemaphoreType.DMA,
        pltpu.SemaphoreType.DMA,
    ],
)
def _emb_sum_kernel(w_hbm, idx_hbm, out_hbm,
                    idxv, a0, a1, a2, a3, b0, b1, b2, b3, ova, ovb,
                    gsa, gsb, osa, osb):
    wid = lax.axis_index("s") * NC + lax.axis_index("c")
    base0 = wid * TOK_PER_W
    sets = (
        ((a0, a1, a2, a3), ova, gsa, osa),
        ((b0, b1, b2, b3), ovb, gsb, osb),
    )

    for k in range(4):
        pltpu.sync_copy(idx_hbm.at[k, pl.ds(base0, TOK_PER_W)], idxv.at[k])

    def issue_gathers(ci, sl):
        bufs, _, gsem, _ = sets[sl]
        off = ci * C
        for k in range(4):
            pltpu.async_copy(w_hbm.at[idxv.at[k, pl.ds(off, C)]], bufs[k], gsem)

    def proc(ci, sl, wait_out, issue_next):
        bufs, ov, gsem, osem = sets[sl]
        off = ci * C
        for k in range(4):
            pltpu.make_async_copy(
                w_hbm.at[idxv.at[k, pl.ds(off, C)]], bufs[k], gsem).wait()
        if wait_out:
            pltpu.make_async_copy(
                ov, out_hbm.at[pl.ds(base0, C)], osem).wait()

        def row_body(r, c2):
            for j in range(8):
                cs = pl.ds(j * 16, 16)
                s0 = bufs[0][r, cs] + bufs[1][r, cs]
                s1 = bufs[2][r, cs] + bufs[3][r, cs]
                ov[r, cs] = s0 + s1
            return c2

        lax.fori_loop(0, C, row_body, 0)
        pltpu.async_copy(ov, out_hbm.at[pl.ds(base0 + off, C)], osem)
        if issue_next:
            issue_gathers(ci + 2, sl)

    issue_gathers(0, 0)
    issue_gathers(1, 1)
    proc(0, 0, wait_out=False, issue_next=True)
    proc(1, 1, wait_out=False, issue_next=True)

    def body(i, c):
        ci = 2 * i
        proc(ci, 0, wait_out=True, issue_next=True)
        proc(ci + 1, 1, wait_out=True, issue_next=True)
        return c

    lax.fori_loop(1, NCHUNK // 2 - 1, body, 0)
    proc(NCHUNK - 2, 0, wait_out=True, issue_next=False)
    proc(NCHUNK - 1, 1, wait_out=True, issue_next=False)
    for _, ov, _, osem in sets:
        pltpu.make_async_copy(ov, out_hbm.at[pl.ds(base0, C)], osem).wait()


def kernel(x, W_sy, W_tone, W_syllable_flag, W_ws):
    w_cat = jnp.concatenate([W_sy, W_tone, W_syllable_flag, W_ws], axis=0)
    idx = x.reshape(N, 4) + jnp.arange(4, dtype=jnp.int32) * VOCAB
    idx_all = idx.T
    out = _emb_sum_kernel(w_cat, idx_all)
    return out.reshape(B, T, D)


# speedup vs baseline: 13.2680x; 1.2115x over previous
import functools

import jax
import jax.numpy as jnp
from jax import lax
from jax.experimental import pallas as pl
from jax.experimental.pallas import tpu as pltpu
from jax.experimental.pallas import tpu_sc as plsc

B, T = 1024, 200
VOCAB = 1024
D = 128
N = B * T
NC, NS = 2, 16
NW = NC * NS
TOK_PER_W = N // NW
C = 64
NCHUNK = TOK_PER_W // C
NHALF = 2
CHUNK_H = NCHUNK // NHALF
TOK_H = TOK_PER_W // NHALF
ROWS_PER_TILE = 4 * VOCAB // NS

_mesh = plsc.VectorSubcoreMesh(core_axis_name="c", subcore_axis_name="s")


@functools.partial(
    pl.kernel,
    mesh=_mesh,
    out_type=jax.ShapeDtypeStruct((N, D), jnp.float32),
    scratch_types=[
        pltpu.VMEM_SHARED((4 * VOCAB, D), jnp.float32),
        pltpu.VMEM((4, TOK_H), jnp.int32),
        pltpu.VMEM((C, D), jnp.float32),
        pltpu.VMEM((C, D), jnp.float32),
        pltpu.VMEM((C, D), jnp.float32),
        pltpu.VMEM((C, D), jnp.float32),
        pltpu.VMEM((C, D), jnp.float32),
        pltpu.VMEM((C, D), jnp.float32),
        pltpu.VMEM((C, D), jnp.float32),
        pltpu.VMEM((C, D), jnp.float32),
        pltpu.VMEM((C, D), jnp.float32),
        pltpu.VMEM((C, D), jnp.float32),
        pltpu.SemaphoreType.DMA,
        pltpu.SemaphoreType.DMA,
        pltpu.SemaphoreType.DMA,
        pltpu.SemaphoreType.DMA,
    ],
)
def _emb_sum_kernel(w_hbm, idx_hbm, out_hbm,
                    w_sh, idxv, a0, a1, a2, a3, b0, b1, b2, b3, ova, ovb,
                    gsa, gsb, osa, osb):
    cid = lax.axis_index("c")
    sid = lax.axis_index("s")
    wid = sid * NC + cid
    base0 = wid * TOK_PER_W
    sets = (
        ((a0, a1, a2, a3), ova, gsa, osa),
        ((b0, b1, b2, b3), ovb, gsb, osb),
    )

    pltpu.sync_copy(w_hbm.at[pl.ds(sid * ROWS_PER_TILE, ROWS_PER_TILE)],
                    w_sh.at[pl.ds(sid * ROWS_PER_TILE, ROWS_PER_TILE)])
    plsc.subcore_barrier()

    def issue_gathers(ci_rel, sl):
        bufs, _, gsem, _ = sets[sl]
        off = ci_rel * C
        for k in range(4):
            pltpu.async_copy(w_sh.at[idxv.at[k, pl.ds(off, C)]], bufs[k], gsem)

    def proc(ci_rel, half_base, sl, wait_out, issue_next):
        bufs, ov, gsem, osem = sets[sl]
        off = ci_rel * C
        for k in range(4):
            pltpu.make_async_copy(
                w_sh.at[idxv.at[k, pl.ds(off, C)]], bufs[k], gsem).wait()
        if wait_out:
            pltpu.make_async_copy(
                ov, out_hbm.at[pl.ds(base0, C)], osem).wait()

        def row_body(r, c2):
            for j in range(8):
                cs = pl.ds(j * 16, 16)
                s0 = bufs[0][r, cs] + bufs[1][r, cs]
                s1 = bufs[2][r, cs] + bufs[3][r, cs]
                ov[r, cs] = s0 + s1
            return c2

        lax.fori_loop(0, C, row_body, 0)
        pltpu.async_copy(ov, out_hbm.at[pl.ds(base0 + half_base + off, C)],
                         osem)
        if issue_next:
            issue_gathers(ci_rel + 2, sl)

    for h in range(NHALF):
        half_base = h * TOK_H
        for k in range(4):
            pltpu.sync_copy(idx_hbm.at[k, pl.ds(base0 + half_base, TOK_H)],
                            idxv.at[k])
        issue_gathers(0, 0)
        issue_gathers(1, 1)
        proc(0, half_base, 0, wait_out=(h > 0), issue_next=True)
        proc(1, half_base, 1, wait_out=(h > 0), issue_next=True)

        def body(i, c):
            ci = 2 * i
            proc(ci, half_base, 0, wait_out=True, issue_next=True)
            proc(ci + 1, half_base, 1, wait_out=True, issue_next=True)
            return c

        lax.fori_loop(1, CHUNK_H // 2 - 1, body, 0)
        proc(CHUNK_H - 2, half_base, 0, wait_out=True, issue_next=False)
        proc(CHUNK_H - 1, half_base, 1, wait_out=True, issue_next=False)

    for _, ov, _, osem in sets:
        pltpu.make_async_copy(ov, out_hbm.at[pl.ds(base0, C)], osem).wait()


def kernel(x, W_sy, W_tone, W_syllable_flag, W_ws):
    w_cat = jnp.concatenate([W_sy, W_tone, W_syllable_flag, W_ws], axis=0)
    idx = x.reshape(N, 4) + jnp.arange(4, dtype=jnp.int32) * VOCAB
    idx_all = idx.T
    out = _emb_sum_kernel(w_cat, idx_all)
    return out.reshape(B, T, D)


# speedup vs baseline: 13.8466x; 1.0436x over previous
import functools

import jax
import jax.numpy as jnp
from jax import lax
from jax.experimental import pallas as pl
from jax.experimental.pallas import tpu as pltpu
from jax.experimental.pallas import tpu_sc as plsc

B, T = 1024, 200
VOCAB = 1024
D = 128
N = B * T
NC, NS = 2, 16
NW = NC * NS
TOK_PER_W = N // NW
C = 64
NCHUNK = TOK_PER_W // C
NHALF = 2
CHUNK_H = NCHUNK // NHALF
TOK_H = TOK_PER_W // NHALF
ROWS_PER_TILE = 4 * VOCAB // NS

_mesh = plsc.VectorSubcoreMesh(core_axis_name="c", subcore_axis_name="s")


@functools.partial(
    pl.kernel,
    mesh=_mesh,
    out_type=jax.ShapeDtypeStruct((N, D), jnp.float32),
    scratch_types=[
        pltpu.VMEM_SHARED((4 * VOCAB, D), jnp.float32),
        pltpu.VMEM((4, TOK_H), jnp.int32),
        pltpu.VMEM((C, D), jnp.float32),
        pltpu.VMEM((C, D), jnp.float32),
        pltpu.VMEM((C, D), jnp.float32),
        pltpu.VMEM((C, D), jnp.float32),
        pltpu.VMEM((C, D), jnp.float32),
        pltpu.VMEM((C, D), jnp.float32),
        pltpu.VMEM((C, D), jnp.float32),
        pltpu.VMEM((C, D), jnp.float32),
        pltpu.VMEM((C, D), jnp.float32),
        pltpu.VMEM((C, D), jnp.float32),
        pltpu.SemaphoreType.DMA,
        pltpu.SemaphoreType.DMA,
        pltpu.SemaphoreType.DMA,
        pltpu.SemaphoreType.DMA,
        pltpu.SemaphoreType.DMA,
        pltpu.SemaphoreType.DMA,
    ],
)
def _emb_sum_kernel(w_hbm, idx_hbm, out_hbm,
                    w_sh, idxv, a0, a1, a2, a3, b0, b1, b2, b3, ova, ovb,
                    gsa, gsb, hsa, hsb, osa, osb):
    cid = lax.axis_index("c")
    sid = lax.axis_index("s")
    wid = sid * NC + cid
    base0 = wid * TOK_PER_W
    sets = (
        ((a0, a1, a2, a3), ova, (gsa, gsa, hsa, hsa), osa),
        ((b0, b1, b2, b3), ovb, (gsb, gsb, hsb, hsb), osb),
    )

    pltpu.sync_copy(w_hbm.at[pl.ds(sid * ROWS_PER_TILE, ROWS_PER_TILE)],
                    w_sh.at[pl.ds(sid * ROWS_PER_TILE, ROWS_PER_TILE)])
    plsc.subcore_barrier()

    w_srcs = (w_sh, w_sh, w_hbm, w_hbm)

    def issue_gathers(ci_rel, sl):
        bufs, _, gsems, _ = sets[sl]
        off = ci_rel * C
        for k in range(4):
            pltpu.async_copy(
                w_srcs[k].at[idxv.at[k, pl.ds(off, C)]], bufs[k], gsems[k])

    def proc(ci_rel, half_base, sl, wait_out, issue_next):
        bufs, ov, gsems, osem = sets[sl]
        off = ci_rel * C
        for k in range(4):
            pltpu.make_async_copy(
                w_srcs[k].at[idxv.at[k, pl.ds(off, C)]],
                bufs[k], gsems[k]).wait()
        if wait_out:
            pltpu.make_async_copy(
                ov, out_hbm.at[pl.ds(base0, C)], osem).wait()

        def row_body(r, c2):
            for j in range(8):
                cs = pl.ds(j * 16, 16)
                s0 = bufs[0][r, cs] + bufs[1][r, cs]
                s1 = bufs[2][r, cs] + bufs[3][r, cs]
                ov[r, cs] = s0 + s1
            return c2

        lax.fori_loop(0, C, row_body, 0)
        pltpu.async_copy(ov, out_hbm.at[pl.ds(base0 + half_base + off, C)],
                         osem)
        if issue_next:
            issue_gathers(ci_rel + 2, sl)

    for h in range(NHALF):
        half_base = h * TOK_H
        for k in range(4):
            pltpu.sync_copy(idx_hbm.at[k, pl.ds(base0 + half_base, TOK_H)],
                            idxv.at[k])
        issue_gathers(0, 0)
        issue_gathers(1, 1)
        proc(0, half_base, 0, wait_out=(h > 0), issue_next=True)
        proc(1, half_base, 1, wait_out=(h > 0), issue_next=True)

        def body(i, c):
            ci = 2 * i
            proc(ci, half_base, 0, wait_out=True, issue_next=True)
            proc(ci + 1, half_base, 1, wait_out=True, issue_next=True)
            return c

        lax.fori_loop(1, CHUNK_H // 2 - 1, body, 0)
        proc(CHUNK_H - 2, half_base, 0, wait_out=True, issue_next=False)
        proc(CHUNK_H - 1, half_base, 1, wait_out=True, issue_next=False)

    for _, ov, _, osem in sets:
        pltpu.make_async_copy(ov, out_hbm.at[pl.ds(base0, C)], osem).wait()


def kernel(x, W_sy, W_tone, W_syllable_flag, W_ws):
    w_cat = jnp.concatenate([W_sy, W_tone, W_syllable_flag, W_ws], axis=0)
    idx = x.reshape(N, 4) + jnp.arange(4, dtype=jnp.int32) * VOCAB
    idx_all = idx.T
    out = _emb_sum_kernel(w_cat, idx_all)
    return out.reshape(B, T, D)


# speedup vs baseline: 14.5889x; 1.0536x over previous
import functools

import jax
import jax.numpy as jnp
from jax import lax
from jax.experimental import pallas as pl
from jax.experimental.pallas import tpu as pltpu
from jax.experimental.pallas import tpu_sc as plsc

B, T = 1024, 200
VOCAB = 1024
D = 128
N = B * T
NC, NS = 2, 16
NW = NC * NS
TOK_PER_W = N // NW
C = 128
NCHUNK = TOK_PER_W // C
NSET = 4
ROWS_PER_TILE = 4 * VOCAB // NS

_mesh = plsc.VectorSubcoreMesh(core_axis_name="c", subcore_axis_name="s")


@functools.partial(
    pl.kernel,
    mesh=_mesh,
    out_type=jax.ShapeDtypeStruct((N, D), jnp.float32),
    scratch_types=[
        pltpu.VMEM_SHARED((4 * VOCAB, D), jnp.float32),
        pltpu.VMEM((4, TOK_PER_W), jnp.int32),
        pltpu.VMEM((C, D), jnp.float32),
        pltpu.VMEM((C, D), jnp.float32),
        pltpu.VMEM((C, D), jnp.float32),
        pltpu.VMEM((C, D), jnp.float32),
        pltpu.SemaphoreType.DMA,
        pltpu.SemaphoreType.DMA,
        pltpu.SemaphoreType.DMA,
        pltpu.SemaphoreType.DMA,
        pltpu.SemaphoreType.DMA,
        pltpu.SemaphoreType.DMA,
        pltpu.SemaphoreType.DMA,
        pltpu.SemaphoreType.DMA,
        pltpu.SemaphoreType.DMA,
        pltpu.SemaphoreType.DMA,
        pltpu.SemaphoreType.DMA,
        pltpu.SemaphoreType.DMA,
    ],
)
def _emb_sum_kernel(w_hbm, idx_hbm, out_hbm,
                    w_sh, idxv, acc0, acc1, acc2, acc3,
                    bs0, bs1, bs2, bs3, as0, as1, as2, as3,
                    os0, os1, os2, os3):
    cid = lax.axis_index("c")
    sid = lax.axis_index("s")
    wid = sid * NC + cid
    base0 = wid * TOK_PER_W
    accs = (acc0, acc1, acc2, acc3)
    bsems = (bs0, bs1, bs2, bs3)
    asems = (as0, as1, as2, as3)
    osems = (os0, os1, os2, os3)

    pltpu.sync_copy(w_hbm.at[pl.ds(sid * ROWS_PER_TILE, ROWS_PER_TILE)],
                    w_sh.at[pl.ds(sid * ROWS_PER_TILE, ROWS_PER_TILE)])
    for k in range(4):
        pltpu.sync_copy(idx_hbm.at[k, pl.ds(base0, TOK_PER_W)], idxv.at[k])
    plsc.subcore_barrier()

    def base_cp(ci, s):
        return pltpu.make_async_copy(
            w_sh.at[idxv.at[0, pl.ds(ci * C, C)]], accs[s], bsems[s])

    def add_start(ci, s, k):
        pltpu.async_copy(
            w_sh.at[idxv.at[k, pl.ds(ci * C, C)]], accs[s], asems[s],
            add=True)

    def add_wait(ci, s, k):
        pltpu.make_async_copy(
            w_sh.at[idxv.at[k, pl.ds(ci * C, C)]], accs[s], asems[s]).wait()

    def out_cp(ci, s):
        return pltpu.make_async_copy(
            accs[s], out_hbm.at[pl.ds(base0 + ci * C, C)], osems[s])

    def step(ci, d, do_a, wait_b, issue_b, do_c):
        if do_a:
            sa = (d - 1) % NSET
            for k in range(1, 4):
                add_wait(ci - 1, sa, k)
            out_cp(ci - 1, sa).start()
        sb = (d + 1) % NSET
        if wait_b:
            out_cp(ci - 3, sb).wait()
        if issue_b:
            base_cp(ci + 1, sb).start()
        if do_c:
            sc_ = d % NSET
            base_cp(ci, sc_).wait()
            for k in range(1, 4):
                add_start(ci, sc_, k)

    base_cp(0, 0).start()
    step(0, 0, do_a=False, wait_b=False, issue_b=True, do_c=True)
    step(1, 1, do_a=True, wait_b=False, issue_b=True, do_c=True)
    step(2, 2, do_a=True, wait_b=False, issue_b=True, do_c=True)
    step(3, 3, do_a=True, wait_b=True, issue_b=True, do_c=True)

    def body(g, carry):
        ci = 4 * g
        for d in range(4):
            step(ci + d, d, do_a=True, wait_b=True, issue_b=True, do_c=True)
        return carry

    lax.fori_loop(1, NCHUNK // 4, body, 0)
    ci_t = NCHUNK - 2
    step(ci_t, ci_t % NSET, do_a=True, wait_b=True, issue_b=True, do_c=True)
    step(ci_t + 1, (ci_t + 1) % NSET, do_a=True, wait_b=True, issue_b=False,
         do_c=True)
    s_last = (NCHUNK - 1) % NSET
    for k in range(1, 4):
        add_wait(NCHUNK - 1, s_last, k)
    out_cp(NCHUNK - 1, s_last).start()
    for ci in (NCHUNK - 3, NCHUNK - 2, NCHUNK - 1):
        out_cp(ci, ci % NSET).wait()


def kernel(x, W_sy, W_tone, W_syllable_flag, W_ws):
    w_cat = jnp.concatenate([W_sy, W_tone, W_syllable_flag, W_ws], axis=0)
    idx = x.reshape(N, 4) + jnp.arange(4, dtype=jnp.int32) * VOCAB
    idx_all = idx.T
    out = _emb_sum_kernel(w_cat, idx_all)
    return out.reshape(B, T, D)


# speedup vs baseline: 15.6403x; 1.0721x over previous
import functools

import jax
import jax.numpy as jnp
from jax import lax
from jax.experimental import pallas as pl
from jax.experimental.pallas import tpu as pltpu
from jax.experimental.pallas import tpu_sc as plsc

B, T = 1024, 200
VOCAB = 1024
D = 128
N = B * T
NC, NS = 2, 16
NW = NC * NS
TOK_PER_W = N // NW
C = 128
NCHUNK = TOK_PER_W // C
NSET = 4
ROWS_PER_TILE = 4 * VOCAB // NS

_mesh = plsc.VectorSubcoreMesh(core_axis_name="c", subcore_axis_name="s")


@functools.partial(
    pl.kernel,
    mesh=_mesh,
    out_type=jax.ShapeDtypeStruct((N, D), jnp.float32),
    scratch_types=[
        pltpu.VMEM_SHARED((4 * VOCAB, D), jnp.float32),
        pltpu.VMEM((4, TOK_PER_W), jnp.int32),
        pltpu.VMEM((C, D), jnp.float32),
        pltpu.VMEM((C, D), jnp.float32),
        pltpu.VMEM((C, D), jnp.float32),
        pltpu.VMEM((C, D), jnp.float32),
        pltpu.SemaphoreType.DMA,
        pltpu.SemaphoreType.DMA,
        pltpu.SemaphoreType.DMA,
        pltpu.SemaphoreType.DMA,
        pltpu.SemaphoreType.DMA,
        pltpu.SemaphoreType.DMA,
        pltpu.SemaphoreType.DMA,
        pltpu.SemaphoreType.DMA,
        pltpu.SemaphoreType.DMA,
        pltpu.SemaphoreType.DMA,
        pltpu.SemaphoreType.DMA,
        pltpu.SemaphoreType.DMA,
        pltpu.SemaphoreType.DMA,
        pltpu.SemaphoreType.DMA,
        pltpu.SemaphoreType.DMA,
        pltpu.SemaphoreType.DMA,
    ],
)
def _emb_sum_kernel(w_hbm, idx_hbm, out_hbm,
                    w_sh, idxv, acc0, acc1, acc2, acc3,
                    bs0, bs1, bs2, bs3, as0, as1, as2, as3,
                    os0, os1, os2, os3, hs0, hs1, hs2, hs3):
    cid = lax.axis_index("c")
    sid = lax.axis_index("s")
    wid = sid * NC + cid
    base0 = wid * TOK_PER_W
    accs = (acc0, acc1, acc2, acc3)
    bsems = (bs0, bs1, bs2, bs3)
    asems = (as0, as1, as2, as3)
    osems = (os0, os1, os2, os3)
    hsems = (hs0, hs1, hs2, hs3)
    w_srcs = (w_sh, w_sh, w_sh, w_hbm)

    pltpu.sync_copy(w_hbm.at[pl.ds(sid * ROWS_PER_TILE, ROWS_PER_TILE)],
                    w_sh.at[pl.ds(sid * ROWS_PER_TILE, ROWS_PER_TILE)])
    for k in range(4):
        pltpu.sync_copy(idx_hbm.at[k, pl.ds(base0, TOK_PER_W)], idxv.at[k])
    plsc.subcore_barrier()

    def base_cp(ci, s):
        return pltpu.make_async_copy(
            w_sh.at[idxv.at[0, pl.ds(ci * C, C)]], accs[s], bsems[s])

    def add_start(ci, s, k):
        sem = hsems[s] if k == 3 else asems[s]
        pltpu.async_copy(
            w_srcs[k].at[idxv.at[k, pl.ds(ci * C, C)]], accs[s], sem,
            add=True)

    def add_wait(ci, s, k):
        sem = hsems[s] if k == 3 else asems[s]
        pltpu.make_async_copy(
            w_srcs[k].at[idxv.at[k, pl.ds(ci * C, C)]], accs[s], sem).wait()

    def out_cp(ci, s):
        return pltpu.make_async_copy(
            accs[s], out_hbm.at[pl.ds(base0 + ci * C, C)], osems[s])

    def step(ci, d, do_a, wait_b, issue_b, do_c):
        if do_a:
            sa = (d - 1) % NSET
            for k in range(1, 4):
                add_wait(ci - 1, sa, k)
            out_cp(ci - 1, sa).start()
        sb = (d + 1) % NSET
        if wait_b:
            out_cp(ci - 3, sb).wait()
        if issue_b:
            base_cp(ci + 1, sb).start()
        if do_c:
            sc_ = d % NSET
            base_cp(ci, sc_).wait()
            for k in range(1, 4):
                add_start(ci, sc_, k)

    base_cp(0, 0).start()
    step(0, 0, do_a=False, wait_b=False, issue_b=True, do_c=True)
    step(1, 1, do_a=True, wait_b=False, issue_b=True, do_c=True)
    step(2, 2, do_a=True, wait_b=False, issue_b=True, do_c=True)
    step(3, 3, do_a=True, wait_b=True, issue_b=True, do_c=True)

    def body(g, carry):
        ci = 4 * g
        for d in range(4):
            step(ci + d, d, do_a=True, wait_b=True, issue_b=True, do_c=True)
        return carry

    lax.fori_loop(1, NCHUNK // 4, body, 0)
    ci_t = NCHUNK - 2
    step(ci_t, ci_t % NSET, do_a=True, wait_b=True, issue_b=True, do_c=True)
    step(ci_t + 1, (ci_t + 1) % NSET, do_a=True, wait_b=True, issue_b=False,
         do_c=True)
    s_last = (NCHUNK - 1) % NSET
    for k in range(1, 4):
        add_wait(NCHUNK - 1, s_last, k)
    out_cp(NCHUNK - 1, s_last).start()
    for ci in (NCHUNK - 3, NCHUNK - 2, NCHUNK - 1):
        out_cp(ci, ci % NSET).wait()


def kernel(x, W_sy, W_tone, W_syllable_flag, W_ws):
    w_cat = jnp.concatenate([W_sy, W_tone, W_syllable_flag, W_ws], axis=0)
    idx = x.reshape(N, 4) + jnp.arange(4, dtype=jnp.int32) * VOCAB
    idx_all = idx.T
    out = _emb_sum_kernel(w_cat, idx_all)
    return out.reshape(B, T, D)
